# baseline (speedup 1.0000x reference)
import functools

import jax
import jax.numpy as jnp
from jax import lax
from jax.experimental import pallas as pl
from jax.experimental.pallas import tpu as pltpu
from jax.experimental.pallas import tpu_sc as plsc

_RES_X = 1024
_RES_Y = 1024
_DH = 256
_B = 16 * 32 * 128
_NC = 2
_NS = 16
_L = 16
_NW = _NC * _NS
_BPW = _B // _NW
_CHUNK = 64
_NCHUNK = _BPW // _CHUNK
_ZROW = _RES_X


def _pos_enc_body(cx_hbm, cy_hbm, sz_hbm, xt_hbm, yt_hbm, out_hbm,
                  s_v, cxa, cya, ixa, iya,
                  xr0, yr0, xr1, yr1, xr2, yr2,
                  sx0, sy0, sx1, sy1, sx2, sy2):
    wid = lax.axis_index("s") * _NC + lax.axis_index("c")
    base = wid * _BPW
    crow = pl.multiple_of(wid * _NCHUNK, _NCHUNK)
    pltpu.sync_copy(cx_hbm.at[pl.ds(crow, _NCHUNK)], cxa)
    pltpu.sync_copy(cy_hbm.at[pl.ds(crow, _NCHUNK)], cya)
    pltpu.sync_copy(sz_hbm, s_v)
    s_h = s_v[0, :]
    s_w = s_v[1, :]

    for k in range(_NCHUNK):
        for i in range(_CHUNK // _L):
            sl = pl.ds(i * _L, _L)
            x = cxa[k, sl]
            y = cya[k, sl]
            ix = jnp.clip(((_RES_X * x) / s_w).astype(jnp.int32), 0, _RES_X - 1)
            iy = jnp.clip(((_RES_Y * y) / s_h).astype(jnp.int32), 0, _RES_Y - 1)
            neg = x < 0.0
            ixa[k, sl] = jnp.where(neg, _ZROW, ix)
            iya[k, sl] = jnp.where(neg, _ZROW, iy)

    slots = ((xr0, yr0, sx0, sy0), (xr1, yr1, sx1, sy1), (xr2, yr2, sx2, sy2))

    def fire(slot, k):
        xr_v, yr_v, semx, semy = slot
        pltpu.async_copy(xt_hbm.at[ixa.at[k]], xr_v, semx)
        pltpu.async_copy(yt_hbm.at[iya.at[k]], yr_v, semy)

    def drain(slot, k):
        xr_v, yr_v, semx, semy = slot
        off = pl.multiple_of(base + k * _CHUNK, _CHUNK)
        pltpu.make_async_copy(xt_hbm.at[ixa.at[k]], xr_v, semx).wait()
        pltpu.sync_copy(xr_v, out_hbm.at[pl.ds(off, _CHUNK), pl.ds(0, _DH)])
        pltpu.make_async_copy(yt_hbm.at[iya.at[k]], yr_v, semy).wait()
        pltpu.sync_copy(yr_v, out_hbm.at[pl.ds(off, _CHUNK), pl.ds(_DH, _DH)])

    fire(slots[0], 0)
    fire(slots[1], 1)

    def body(m, carry):
        k = 3 * m
        fire(slots[2], k + 2)
        drain(slots[0], k)
        fire(slots[0], k + 3)
        drain(slots[1], k + 1)
        fire(slots[1], k + 4)
        drain(slots[2], k + 2)
        return carry

    lax.fori_loop(0, _NCHUNK // 3, body, 0)
    drain(slots[0], _NCHUNK - 2)
    drain(slots[1], _NCHUNK - 1)


_pos_enc = functools.partial(
    pl.kernel,
    out_type=jax.ShapeDtypeStruct((_B, 2 * _DH), jnp.float32),
    mesh=plsc.VectorSubcoreMesh(core_axis_name="c", subcore_axis_name="s"),
    scratch_types=[
        pltpu.VMEM((2, _L), jnp.float32),
        pltpu.VMEM((_NCHUNK, _CHUNK), jnp.float32),
        pltpu.VMEM((_NCHUNK, _CHUNK), jnp.float32),
        pltpu.VMEM((_NCHUNK, _CHUNK), jnp.int32),
        pltpu.VMEM((_NCHUNK, _CHUNK), jnp.int32),
        pltpu.VMEM((_CHUNK, _DH), jnp.float32),
        pltpu.VMEM((_CHUNK, _DH), jnp.float32),
        pltpu.VMEM((_CHUNK, _DH), jnp.float32),
        pltpu.VMEM((_CHUNK, _DH), jnp.float32),
        pltpu.VMEM((_CHUNK, _DH), jnp.float32),
        pltpu.VMEM((_CHUNK, _DH), jnp.float32),
        pltpu.SemaphoreType.DMA,
        pltpu.SemaphoreType.DMA,
        pltpu.SemaphoreType.DMA,
        pltpu.SemaphoreType.DMA,
        pltpu.SemaphoreType.DMA,
        pltpu.SemaphoreType.DMA,
    ],
)(_pos_enc_body)


def kernel(coordinate, size, x_embedding, y_embedding):
    lead = coordinate.shape[:-1]
    cx = coordinate[..., 0].reshape(_B // _CHUNK, _CHUNK)
    cy = coordinate[..., 1].reshape(_B // _CHUNK, _CHUNK)
    zrow = jnp.zeros((8, _DH), x_embedding.dtype)
    xt = jnp.concatenate([x_embedding, zrow], axis=0)
    yt = jnp.concatenate([y_embedding, zrow], axis=0)
    svec = jnp.broadcast_to(size.astype(jnp.float32).reshape(2, 1), (2, _L))
    out = _pos_enc(cx, cy, svec, xt, yt)
    return out.reshape(*lead, 2 * _DH)


# speedup vs baseline: 1.0011x; 1.0011x over previous
import functools

import jax
import jax.numpy as jnp
from jax import lax
from jax.experimental import pallas as pl
from jax.experimental.pallas import tpu as pltpu
from jax.experimental.pallas import tpu_sc as plsc

_RES_X = 1024
_RES_Y = 1024
_DH = 256
_B = 16 * 32 * 128
_NC = 2
_NS = 16
_L = 16
_NW = _NC * _NS
_BPW = _B // _NW
_CHUNK = 64
_NCHUNK = _BPW // _CHUNK
_ZROW = _RES_X


def _pos_enc_body(cx_hbm, cy_hbm, sz_hbm, xt_hbm, yt_hbm, out_hbm,
                  s_v, cxa, cya, ixa, iya,
                  xr0, yr0, xr1, yr1, xr2, yr2,
                  sx0, sy0, sx1, sy1, sx2, sy2,
                  wx0, wy0, wx1, wy1, wx2, wy2):
    wid = lax.axis_index("s") * _NC + lax.axis_index("c")
    base = wid * _BPW
    crow = pl.multiple_of(wid * _NCHUNK, _NCHUNK)
    pltpu.sync_copy(cx_hbm.at[pl.ds(crow, _NCHUNK)], cxa)
    pltpu.sync_copy(cy_hbm.at[pl.ds(crow, _NCHUNK)], cya)
    pltpu.sync_copy(sz_hbm, s_v)
    s_h = s_v[0, :]
    s_w = s_v[1, :]

    for k in range(_NCHUNK):
        for i in range(_CHUNK // _L):
            sl = pl.ds(i * _L, _L)
            x = cxa[k, sl]
            y = cya[k, sl]
            ix = jnp.clip(((_RES_X * x) / s_w).astype(jnp.int32), 0, _RES_X - 1)
            iy = jnp.clip(((_RES_Y * y) / s_h).astype(jnp.int32), 0, _RES_Y - 1)
            neg = x < 0.0
            ixa[k, sl] = jnp.where(neg, _ZROW, ix)
            iya[k, sl] = jnp.where(neg, _ZROW, iy)

    slots = ((xr0, yr0, sx0, sy0, wx0, wy0),
             (xr1, yr1, sx1, sy1, wx1, wy1),
             (xr2, yr2, sx2, sy2, wx2, wy2))

    def fire_g(slot, k):
        xr_v, yr_v, semx, semy, _, _ = slot
        pltpu.async_copy(xt_hbm.at[ixa.at[k]], xr_v, semx)
        pltpu.async_copy(yt_hbm.at[iya.at[k]], yr_v, semy)

    def wait_g_fire_w(slot, k):
        xr_v, yr_v, semx, semy, semwx, semwy = slot
        off = pl.multiple_of(base + k * _CHUNK, _CHUNK)
        pltpu.make_async_copy(xt_hbm.at[ixa.at[k]], xr_v, semx).wait()
        pltpu.async_copy(xr_v, out_hbm.at[pl.ds(off, _CHUNK), pl.ds(0, _DH)],
                         semwx)
        pltpu.make_async_copy(yt_hbm.at[iya.at[k]], yr_v, semy).wait()
        pltpu.async_copy(yr_v, out_hbm.at[pl.ds(off, _CHUNK), pl.ds(_DH, _DH)],
                         semwy)

    def wait_w(slot, k):
        xr_v, yr_v, _, _, semwx, semwy = slot
        off = pl.multiple_of(base + k * _CHUNK, _CHUNK)
        pltpu.make_async_copy(
            xr_v, out_hbm.at[pl.ds(off, _CHUNK), pl.ds(0, _DH)], semwx).wait()
        pltpu.make_async_copy(
            yr_v, out_hbm.at[pl.ds(off, _CHUNK), pl.ds(_DH, _DH)], semwy).wait()

    fire_g(slots[0], 0)
    fire_g(slots[1], 1)
    fire_g(slots[2], 2)
    wait_g_fire_w(slots[0], 0)
    wait_g_fire_w(slots[1], 1)
    wait_w(slots[0], 0); fire_g(slots[0], 3)
    wait_g_fire_w(slots[2], 2)
    wait_w(slots[1], 1); fire_g(slots[1], 4)

    def body(m, carry):
        k = 3 * m
        wait_g_fire_w(slots[0], k)
        wait_w(slots[2], k - 1); fire_g(slots[2], k + 2)
        wait_g_fire_w(slots[1], k + 1)
        wait_w(slots[0], k); fire_g(slots[0], k + 3)
        wait_g_fire_w(slots[2], k + 2)
        wait_w(slots[1], k + 1); fire_g(slots[1], k + 4)
        return carry

    lax.fori_loop(1, _NCHUNK // 3, body, 0)
    wait_g_fire_w(slots[0], _NCHUNK - 2)
    wait_w(slots[2], _NCHUNK - 3)
    wait_g_fire_w(slots[1], _NCHUNK - 1)
    wait_w(slots[0], _NCHUNK - 2)
    wait_w(slots[1], _NCHUNK - 1)


_pos_enc = functools.partial(
    pl.kernel,
    out_type=jax.ShapeDtypeStruct((_B, 2 * _DH), jnp.float32),
    mesh=plsc.VectorSubcoreMesh(core_axis_name="c", subcore_axis_name="s"),
    scratch_types=[
        pltpu.VMEM((2, _L), jnp.float32),
        pltpu.VMEM((_NCHUNK, _CHUNK), jnp.float32),
        pltpu.VMEM((_NCHUNK, _CHUNK), jnp.float32),
        pltpu.VMEM((_NCHUNK, _CHUNK), jnp.int32),
        pltpu.VMEM((_NCHUNK, _CHUNK), jnp.int32),
        pltpu.VMEM((_CHUNK, _DH), jnp.float32),
        pltpu.VMEM((_CHUNK, _DH), jnp.float32),
        pltpu.VMEM((_CHUNK, _DH), jnp.float32),
        pltpu.VMEM((_CHUNK, _DH), jnp.float32),
        pltpu.VMEM((_CHUNK, _DH), jnp.float32),
        pltpu.VMEM((_CHUNK, _DH), jnp.float32),
        pltpu.SemaphoreType.DMA,
        pltpu.SemaphoreType.DMA,
        pltpu.SemaphoreType.DMA,
        pltpu.SemaphoreType.DMA,
        pltpu.SemaphoreType.DMA,
        pltpu.SemaphoreType.DMA,
        pltpu.SemaphoreType.DMA,
        pltpu.SemaphoreType.DMA,
        pltpu.SemaphoreType.DMA,
        pltpu.SemaphoreType.DMA,
        pltpu.SemaphoreType.DMA,
        pltpu.SemaphoreType.DMA,
    ],
)(_pos_enc_body)


def kernel(coordinate, size, x_embedding, y_embedding):
    lead = coordinate.shape[:-1]
    cx = coordinate[..., 0].reshape(_B // _CHUNK, _CHUNK)
    cy = coordinate[..., 1].reshape(_B // _CHUNK, _CHUNK)
    zrow = jnp.zeros((8, _DH), x_embedding.dtype)
    xt = jnp.concatenate([x_embedding, zrow], axis=0)
    yt = jnp.concatenate([y_embedding, zrow], axis=0)
    svec = jnp.broadcast_to(size.astype(jnp.float32).reshape(2, 1), (2, _L))
    out = _pos_enc(cx, cy, svec, xt, yt)
    return out.reshape(*lead, 2 * _DH)
